# baseline (speedup 1.0000x reference)
import functools

import numpy as np
import jax
import jax.numpy as jnp
from jax import lax
from jax.experimental import pallas as pl
from jax.experimental.pallas import tpu as pltpu
from jax.experimental.pallas import tpu_sc as plsc

_NUM_BITS = 8
_QMAX = 2.0 ** (_NUM_BITS - 1) - 1.0
_QMIN = -(2.0 ** (_NUM_BITS - 1))
_MAX_P = 0.999999
_MIN_P = 0.999999
_STEPS = np.arange(0.7, 1.3, 0.01).astype(np.float32)
_NS = len(_STEPS)
_K = 10

_LANES = 2048
_BLK_ROWS = 256

_SC_ROWS = 1024
_SC_WORKERS = 32
_SC_L = 16
_SC_GROUP = 16
_SC_MAGIC = np.float32(1.5 * 2.0**23)


def _quantile_weights(n: int):
  n1 = np.float32(n) - np.float32(1.0)
  qh = np.float32(_MAX_P) * n1
  ql = np.float32(1.0 - _MIN_P) * n1
  out = {}
  out["hi_low_rank"] = n - 1 - int(np.floor(qh))
  out["hi_high_rank"] = n - 1 - int(np.ceil(qh))
  out["hi_hw"] = np.float32(qh - np.floor(qh))
  out["hi_lw"] = np.float32(np.float32(1.0) - out["hi_hw"])
  out["lo_low_rank"] = int(np.floor(ql))
  out["lo_high_rank"] = int(np.ceil(ql))
  out["lo_hw"] = np.float32(ql - np.floor(ql))
  out["lo_lw"] = np.float32(np.float32(1.0) - out["lo_hw"])
  return out


def _ranked_value(cand, want_ranks):
  thresh = jnp.float32(jnp.inf)
  rank = jnp.int32(0)
  got = [jnp.float32(0.0) for _ in want_ranks]
  for _ in range(_K):
    cur = jnp.max(jnp.where(cand < thresh, cand, -jnp.inf))
    c = jnp.sum((cand == cur).astype(jnp.int32))
    for i, wr in enumerate(want_ranks):
      hit = jnp.logical_and(rank <= wr, wr < rank + c)
      got[i] = jnp.where(hit, cur, got[i])
    rank = rank + c
    thresh = cur
  return got


def _extremes_kernel(x_ref, sm_ref, top_ref, bot_ref, *, nsteps, qw):
  i = pl.program_id(0)

  @pl.when(i == 0)
  def _init():
    top_ref[...] = jnp.full_like(top_ref, -jnp.inf)
    bot_ref[...] = jnp.full_like(bot_ref, jnp.inf)

  top = [top_ref[k] for k in range(_K)]
  bot = [bot_ref[k] for k in range(_K)]
  for j in range(_BLK_ROWS // 8):
    v = x_ref[pl.ds(j * 8, 8), :]
    w = v
    for k in range(_K):
      o = top[k]
      top[k] = jnp.maximum(o, v)
      v = jnp.minimum(o, v)
    for k in range(_K):
      o = bot[k]
      bot[k] = jnp.minimum(o, w)
      w = jnp.maximum(o, w)
  for k in range(_K):
    top_ref[k] = top[k]
    bot_ref[k] = bot[k]

  @pl.when(i == nsteps - 1)
  def _final():
    tcand = top_ref[...]
    t_low, t_high = _ranked_value(
        tcand, [qw["hi_low_rank"], qw["hi_high_rank"]])
    bcand = -bot_ref[...]
    b_low, b_high = _ranked_value(
        bcand, [qw["lo_low_rank"], qw["lo_high_rank"]])
    cmax = t_low * qw["hi_lw"] + t_high * qw["hi_hw"]
    cmin = (-b_low) * qw["lo_lw"] + (-b_high) * qw["lo_hw"]
    amax = jnp.maximum(jnp.abs(cmax), jnp.abs(cmin))
    thresh = amax * jnp.float32(_STEPS[0])
    risky = jnp.logical_or(jnp.max(top_ref[_K - 1]) >= thresh,
                           jnp.min(bot_ref[_K - 1]) <= -thresh)
    sm_ref[0] = cmax
    sm_ref[1] = cmin
    sm_ref[2] = amax
    sm_ref[3] = jnp.where(risky, jnp.float32(1.0), jnp.float32(0.0))


def _sweep_base_kernel(invs_ref, x_ref, base_ref, *, nsteps):
  i = pl.program_id(0)

  @pl.when(i == 0)
  def _init():
    for r in range(_NS):
      base_ref[r] = jnp.float32(0.0)

  x = x_ref[...]
  for r in range(_NS):
    u = x * invs_ref[r]
    d = jnp.round(u) - u
    base_ref[r] = base_ref[r] + jnp.sum(d * d)


def _sc_shard_elems():
  return _SC_ROWS * _LANES // _SC_WORKERS


def _sc_sweep_body(x_hbm, invs_hbm, out_hbm, xv, invv, stage):
  shard = _sc_shard_elems()
  wid = lax.axis_index("s") * 2 + lax.axis_index("c")
  base = pl.multiple_of(wid * shard, 8)
  pltpu.sync_copy(x_hbm.at[pl.ds(base, shard)], xv)
  pltpu.sync_copy(invs_hbm, invv)
  nv = shard // _SC_L
  magic = jnp.full((_SC_L,), _SC_MAGIC, jnp.float32)
  for g0 in range(0, _NS, _SC_GROUP):
    gsz = min(_SC_GROUP, _NS - g0)
    inv_vecs = [
        invv[pl.ds((g0 + j) * _SC_L, _SC_L)] for j in range(gsz)
    ]

    def body(i, accs, _inv_vecs=inv_vecs, _gsz=gsz):
      v0 = xv[pl.ds(pl.multiple_of(i * (2 * _SC_L), _SC_L), _SC_L)]
      v1 = xv[pl.ds(pl.multiple_of(i * (2 * _SC_L) + _SC_L, _SC_L), _SC_L)]
      new = []
      for j in range(_gsz):
        inv = _inv_vecs[j]
        u0 = v0 * inv
        u1 = v1 * inv
        d0 = ((u0 + magic) - magic) - u0
        d1 = ((u1 + magic) - magic) - u1
        new.append(accs[j] + (d0 * d0 + d1 * d1))
      return tuple(new)

    accs = lax.fori_loop(
        0, nv // 2, body,
        tuple(jnp.zeros((_SC_L,), jnp.float32) for _ in range(gsz)))
    for j in range(gsz):
      stage[pl.ds((g0 + j) * _SC_L, _SC_L)] = accs[j]
  pltpu.sync_copy(stage, out_hbm.at[wid])


def _finish_kernel(invs_ref, scales_ref, clips_ref, base_ref, scpart_ref,
                   top_ref, bot_ref, out_ref):
  cand = [top_ref[...], bot_ref[...]]
  best_loss = jnp.float32(jnp.inf)
  best_scale = jnp.float32(0.0)
  best_clip = jnp.float32(0.0)
  for r in range(_NS):
    inv = invs_ref[r]
    corr = jnp.sum(scpart_ref[:, r * _SC_L:(r + 1) * _SC_L])
    for v in cand:
      u = v * inv
      rq = jnp.round(u)
      du = rq - u
      dc = jnp.clip(rq, _QMIN, _QMAX) - u
      corr = corr + jnp.sum(dc * dc - du * du)
    s = scales_ref[r]
    loss = (base_ref[r] + corr) * (s * s)
    take = loss < best_loss
    best_loss = jnp.where(take, loss, best_loss)
    best_scale = jnp.where(take, s, best_scale)
    best_clip = jnp.where(take, clips_ref[r], best_clip)
  out_ref[0] = best_scale
  out_ref[1] = jnp.float32(0.0)
  out_ref[2] = best_clip
  out_ref[3] = -best_clip


def _sweep_exact_kernel(invs_ref, scales_ref, clips_ref, x_ref, out_ref,
                        acc_ref, *, nsteps):
  i = pl.program_id(0)

  @pl.when(i == 0)
  def _init():
    for r in range(_NS):
      acc_ref[r] = jnp.float32(0.0)

  x = x_ref[...]
  for r in range(_NS):
    u = x * invs_ref[r]
    d = jnp.clip(jnp.round(u), _QMIN, _QMAX) - u
    acc_ref[r] = acc_ref[r] + jnp.sum(d * d)

  @pl.when(i == nsteps - 1)
  def _final():
    best_loss = jnp.float32(jnp.inf)
    best_scale = jnp.float32(0.0)
    best_clip = jnp.float32(0.0)
    for r in range(_NS):
      s = scales_ref[r]
      loss = acc_ref[r] * (s * s)
      take = loss < best_loss
      best_loss = jnp.where(take, loss, best_loss)
      best_scale = jnp.where(take, s, best_scale)
      best_clip = jnp.where(take, clips_ref[r], best_clip)
    out_ref[0] = best_scale
    out_ref[1] = jnp.float32(0.0)
    out_ref[2] = best_clip
    out_ref[3] = -best_clip


@jax.jit
def kernel(inputs):
  x = inputs.astype(jnp.float32).reshape(-1, _LANES)
  rows = x.shape[0]
  nsteps = rows // _BLK_ROWS
  qw = _quantile_weights(rows * _LANES)

  ext, top, bot = pl.pallas_call(
      functools.partial(_extremes_kernel, nsteps=nsteps, qw=qw),
      grid=(nsteps,),
      in_specs=[pl.BlockSpec((_BLK_ROWS, _LANES), lambda i: (i, 0))],
      out_specs=[
          pl.BlockSpec(memory_space=pltpu.SMEM),
          pl.BlockSpec((_K, 8, _LANES), lambda i: (0, 0, 0)),
          pl.BlockSpec((_K, 8, _LANES), lambda i: (0, 0, 0)),
      ],
      out_shape=[
          jax.ShapeDtypeStruct((4,), jnp.float32),
          jax.ShapeDtypeStruct((_K, 8, _LANES), jnp.float32),
          jax.ShapeDtypeStruct((_K, 8, _LANES), jnp.float32),
      ],
      compiler_params=pltpu.CompilerParams(
          dimension_semantics=("arbitrary",)),
  )(x)

  amax = ext[2]
  risky = ext[3] > jnp.float32(0.5)
  steps = jnp.asarray(_STEPS)
  clips = amax * steps
  scales = clips / jnp.float32(_QMAX)
  invs = jnp.float32(1.0) / scales

  smem_spec = pl.BlockSpec(memory_space=pltpu.SMEM)
  tc_rows = rows - _SC_ROWS
  tc_steps = tc_rows // _BLK_ROWS
  invs_bcast = jnp.repeat(invs, _SC_L)

  def _fast(xx):
    x_tc = xx[:tc_rows]
    x_sc = xx[tc_rows:].reshape(-1)
    scpart = pl.kernel(
        _sc_sweep_body,
        mesh=plsc.VectorSubcoreMesh(core_axis_name="c",
                                    subcore_axis_name="s"),
        out_type=jax.ShapeDtypeStruct((_SC_WORKERS, _NS * _SC_L),
                                      jnp.float32),
        scratch_types=[
            pltpu.VMEM((_sc_shard_elems(),), jnp.float32),
            pltpu.VMEM((_NS * _SC_L,), jnp.float32),
            pltpu.VMEM((_NS * _SC_L,), jnp.float32),
        ],
    )(x_sc, invs_bcast)
    base = pl.pallas_call(
        functools.partial(_sweep_base_kernel, nsteps=tc_steps),
        grid=(tc_steps,),
        in_specs=[smem_spec,
                  pl.BlockSpec((_BLK_ROWS, _LANES), lambda i: (i, 0))],
        out_specs=smem_spec,
        out_shape=jax.ShapeDtypeStruct((_NS,), jnp.float32),
        compiler_params=pltpu.CompilerParams(
            dimension_semantics=("arbitrary",)),
    )(invs, x_tc)
    return pl.pallas_call(
        _finish_kernel,
        in_specs=[smem_spec, smem_spec, smem_spec, smem_spec,
                  pl.BlockSpec((_SC_WORKERS, _NS * _SC_L),
                               lambda: (0, 0)),
                  pl.BlockSpec((_K, 8, _LANES), lambda: (0, 0, 0)),
                  pl.BlockSpec((_K, 8, _LANES), lambda: (0, 0, 0))],
        out_specs=smem_spec,
        out_shape=jax.ShapeDtypeStruct((4,), jnp.float32),
    )(invs, scales, clips, base, scpart, top, bot)

  def _slow(xx):
    return pl.pallas_call(
        functools.partial(_sweep_exact_kernel, nsteps=nsteps),
        grid=(nsteps,),
        in_specs=[smem_spec, smem_spec, smem_spec,
                  pl.BlockSpec((_BLK_ROWS, _LANES), lambda i: (i, 0))],
        out_specs=smem_spec,
        out_shape=jax.ShapeDtypeStruct((4,), jnp.float32),
        scratch_shapes=[pltpu.SMEM((_NS,), jnp.float32)],
        compiler_params=pltpu.CompilerParams(
            dimension_semantics=("arbitrary",)),
    )(invs, scales, clips, xx)

  out = jax.lax.cond(risky, _slow, _fast, x)
  return (out[0].reshape(()), out[1].reshape(()),
          out[2].reshape(()), out[3].reshape(()))


# speedup vs baseline: 1.0804x; 1.0804x over previous
import functools

import numpy as np
import jax
import jax.numpy as jnp
from jax import lax
from jax.experimental import pallas as pl
from jax.experimental.pallas import tpu as pltpu
from jax.experimental.pallas import tpu_sc as plsc

_NUM_BITS = 8
_QMAX = 2.0 ** (_NUM_BITS - 1) - 1.0
_QMIN = -(2.0 ** (_NUM_BITS - 1))
_MAX_P = 0.999999
_MIN_P = 0.999999
_STEPS = np.arange(0.7, 1.3, 0.01).astype(np.float32)
_NS = len(_STEPS)
_K = 10

_LANES = 2048
_BLK_ROWS = 256

_SC_ROWS = 896
_SC_WORKERS = 32
_SC_L = 16
_SC_GROUP = 16
_SC_MAGIC = np.float32(1.5 * 2.0**23)


def _quantile_weights(n: int):
  n1 = np.float32(n) - np.float32(1.0)
  qh = np.float32(_MAX_P) * n1
  ql = np.float32(1.0 - _MIN_P) * n1
  out = {}
  out["hi_low_rank"] = n - 1 - int(np.floor(qh))
  out["hi_high_rank"] = n - 1 - int(np.ceil(qh))
  out["hi_hw"] = np.float32(qh - np.floor(qh))
  out["hi_lw"] = np.float32(np.float32(1.0) - out["hi_hw"])
  out["lo_low_rank"] = int(np.floor(ql))
  out["lo_high_rank"] = int(np.ceil(ql))
  out["lo_hw"] = np.float32(ql - np.floor(ql))
  out["lo_lw"] = np.float32(np.float32(1.0) - out["lo_hw"])
  return out


def _ranked_value(cand, want_ranks):
  thresh = jnp.float32(jnp.inf)
  rank = jnp.int32(0)
  got = [jnp.float32(0.0) for _ in want_ranks]
  for _ in range(_K):
    cur = jnp.max(jnp.where(cand < thresh, cand, -jnp.inf))
    c = jnp.sum((cand == cur).astype(jnp.int32))
    for i, wr in enumerate(want_ranks):
      hit = jnp.logical_and(rank <= wr, wr < rank + c)
      got[i] = jnp.where(hit, cur, got[i])
    rank = rank + c
    thresh = cur
  return got


def _extremes_kernel(x_ref, sm_ref, top_ref, bot_ref, *, nsteps, qw):
  i = pl.program_id(0)

  @pl.when(i == 0)
  def _init():
    top_ref[...] = jnp.full_like(top_ref, -jnp.inf)
    bot_ref[...] = jnp.full_like(bot_ref, jnp.inf)

  top = [top_ref[k] for k in range(_K)]
  bot = [bot_ref[k] for k in range(_K)]
  for j in range(_BLK_ROWS // 8):
    v = x_ref[pl.ds(j * 8, 8), :]
    w = v
    for k in range(_K):
      o = top[k]
      top[k] = jnp.maximum(o, v)
      v = jnp.minimum(o, v)
    for k in range(_K):
      o = bot[k]
      bot[k] = jnp.minimum(o, w)
      w = jnp.maximum(o, w)
  for k in range(_K):
    top_ref[k] = top[k]
    bot_ref[k] = bot[k]

  @pl.when(i == nsteps - 1)
  def _final():
    tcand = top_ref[...]
    t_low, t_high = _ranked_value(
        tcand, [qw["hi_low_rank"], qw["hi_high_rank"]])
    bcand = -bot_ref[...]
    b_low, b_high = _ranked_value(
        bcand, [qw["lo_low_rank"], qw["lo_high_rank"]])
    cmax = t_low * qw["hi_lw"] + t_high * qw["hi_hw"]
    cmin = (-b_low) * qw["lo_lw"] + (-b_high) * qw["lo_hw"]
    amax = jnp.maximum(jnp.abs(cmax), jnp.abs(cmin))
    thresh = amax * jnp.float32(_STEPS[0])
    risky = jnp.logical_or(jnp.max(top_ref[_K - 1]) >= thresh,
                           jnp.min(bot_ref[_K - 1]) <= -thresh)
    sm_ref[0] = cmax
    sm_ref[1] = cmin
    sm_ref[2] = amax
    sm_ref[3] = jnp.where(risky, jnp.float32(1.0), jnp.float32(0.0))


def _sweep_base_kernel(invs_ref, x_ref, base_ref, *, nsteps):
  i = pl.program_id(0)

  @pl.when(i == 0)
  def _init():
    for r in range(_NS):
      base_ref[r] = jnp.float32(0.0)

  x = x_ref[...]
  for r in range(_NS):
    u = x * invs_ref[r]
    d = jnp.round(u) - u
    base_ref[r] = base_ref[r] + jnp.sum(d * d)


def _sc_shard_elems():
  return _SC_ROWS * _LANES // _SC_WORKERS


def _sc_sweep_body(x_hbm, invs_hbm, out_hbm, xv, invv, stage):
  shard = _sc_shard_elems()
  wid = lax.axis_index("s") * 2 + lax.axis_index("c")
  base = pl.multiple_of(wid * shard, 8)
  pltpu.sync_copy(x_hbm.at[pl.ds(base, shard)], xv)
  pltpu.sync_copy(invs_hbm, invv)
  nv = shard // _SC_L
  magic = jnp.full((_SC_L,), _SC_MAGIC, jnp.float32)
  for g0 in range(0, _NS, _SC_GROUP):
    gsz = min(_SC_GROUP, _NS - g0)
    inv_vecs = [
        invv[pl.ds((g0 + j) * _SC_L, _SC_L)] for j in range(gsz)
    ]

    def body(i, accs, _inv_vecs=inv_vecs, _gsz=gsz):
      v0 = xv[pl.ds(pl.multiple_of(i * (2 * _SC_L), _SC_L), _SC_L)]
      v1 = xv[pl.ds(pl.multiple_of(i * (2 * _SC_L) + _SC_L, _SC_L), _SC_L)]
      new = []
      for j in range(_gsz):
        inv = _inv_vecs[j]
        u0 = v0 * inv
        u1 = v1 * inv
        d0 = ((u0 + magic) - magic) - u0
        d1 = ((u1 + magic) - magic) - u1
        new.append(accs[j] + (d0 * d0 + d1 * d1))
      return tuple(new)

    accs = lax.fori_loop(
        0, nv // 2, body,
        tuple(jnp.zeros((_SC_L,), jnp.float32) for _ in range(gsz)))
    for j in range(gsz):
      stage[pl.ds((g0 + j) * _SC_L, _SC_L)] = accs[j]
  pltpu.sync_copy(stage, out_hbm.at[wid])


def _finish_kernel(invs_ref, scales_ref, clips_ref, base_ref, scpart_ref,
                   top_ref, bot_ref, out_ref):
  cand = [top_ref[...], bot_ref[...]]
  best_loss = jnp.float32(jnp.inf)
  best_scale = jnp.float32(0.0)
  best_clip = jnp.float32(0.0)
  for r in range(_NS):
    inv = invs_ref[r]
    corr = jnp.sum(scpart_ref[:, r * _SC_L:(r + 1) * _SC_L])
    for v in cand:
      u = v * inv
      rq = jnp.round(u)
      du = rq - u
      dc = jnp.clip(rq, _QMIN, _QMAX) - u
      corr = corr + jnp.sum(dc * dc - du * du)
    s = scales_ref[r]
    loss = (base_ref[r] + corr) * (s * s)
    take = loss < best_loss
    best_loss = jnp.where(take, loss, best_loss)
    best_scale = jnp.where(take, s, best_scale)
    best_clip = jnp.where(take, clips_ref[r], best_clip)
  out_ref[0] = best_scale
  out_ref[1] = jnp.float32(0.0)
  out_ref[2] = best_clip
  out_ref[3] = -best_clip


def _sweep_exact_kernel(invs_ref, scales_ref, clips_ref, x_ref, out_ref,
                        acc_ref, *, nsteps):
  i = pl.program_id(0)

  @pl.when(i == 0)
  def _init():
    for r in range(_NS):
      acc_ref[r] = jnp.float32(0.0)

  x = x_ref[...]
  for r in range(_NS):
    u = x * invs_ref[r]
    d = jnp.clip(jnp.round(u), _QMIN, _QMAX) - u
    acc_ref[r] = acc_ref[r] + jnp.sum(d * d)

  @pl.when(i == nsteps - 1)
  def _final():
    best_loss = jnp.float32(jnp.inf)
    best_scale = jnp.float32(0.0)
    best_clip = jnp.float32(0.0)
    for r in range(_NS):
      s = scales_ref[r]
      loss = acc_ref[r] * (s * s)
      take = loss < best_loss
      best_loss = jnp.where(take, loss, best_loss)
      best_scale = jnp.where(take, s, best_scale)
      best_clip = jnp.where(take, clips_ref[r], best_clip)
    out_ref[0] = best_scale
    out_ref[1] = jnp.float32(0.0)
    out_ref[2] = best_clip
    out_ref[3] = -best_clip


@jax.jit
def kernel(inputs):
  x = inputs.astype(jnp.float32).reshape(-1, _LANES)
  rows = x.shape[0]
  nsteps = rows // _BLK_ROWS
  qw = _quantile_weights(rows * _LANES)

  ext, top, bot = pl.pallas_call(
      functools.partial(_extremes_kernel, nsteps=nsteps, qw=qw),
      grid=(nsteps,),
      in_specs=[pl.BlockSpec((_BLK_ROWS, _LANES), lambda i: (i, 0))],
      out_specs=[
          pl.BlockSpec(memory_space=pltpu.SMEM),
          pl.BlockSpec((_K, 8, _LANES), lambda i: (0, 0, 0)),
          pl.BlockSpec((_K, 8, _LANES), lambda i: (0, 0, 0)),
      ],
      out_shape=[
          jax.ShapeDtypeStruct((4,), jnp.float32),
          jax.ShapeDtypeStruct((_K, 8, _LANES), jnp.float32),
          jax.ShapeDtypeStruct((_K, 8, _LANES), jnp.float32),
      ],
      compiler_params=pltpu.CompilerParams(
          dimension_semantics=("arbitrary",)),
  )(x)

  amax = ext[2]
  risky = ext[3] > jnp.float32(0.5)
  steps = jnp.asarray(_STEPS)
  clips = amax * steps
  scales = clips / jnp.float32(_QMAX)
  invs = jnp.float32(1.0) / scales

  smem_spec = pl.BlockSpec(memory_space=pltpu.SMEM)
  tc_rows = rows - _SC_ROWS
  sweep_blk = 128
  tc_steps = tc_rows // sweep_blk
  invs_bcast = jnp.repeat(invs, _SC_L)

  def _fast(xx):
    x_tc = xx[:tc_rows]
    x_sc = xx[tc_rows:].reshape(-1)
    scpart = pl.kernel(
        _sc_sweep_body,
        mesh=plsc.VectorSubcoreMesh(core_axis_name="c",
                                    subcore_axis_name="s"),
        out_type=jax.ShapeDtypeStruct((_SC_WORKERS, _NS * _SC_L),
                                      jnp.float32),
        scratch_types=[
            pltpu.VMEM((_sc_shard_elems(),), jnp.float32),
            pltpu.VMEM((_NS * _SC_L,), jnp.float32),
            pltpu.VMEM((_NS * _SC_L,), jnp.float32),
        ],
    )(x_sc, invs_bcast)
    base = pl.pallas_call(
        functools.partial(_sweep_base_kernel, nsteps=tc_steps),
        grid=(tc_steps,),
        in_specs=[smem_spec,
                  pl.BlockSpec((sweep_blk, _LANES), lambda i: (i, 0))],
        out_specs=smem_spec,
        out_shape=jax.ShapeDtypeStruct((_NS,), jnp.float32),
        compiler_params=pltpu.CompilerParams(
            dimension_semantics=("arbitrary",)),
    )(invs, x_tc)
    return pl.pallas_call(
        _finish_kernel,
        in_specs=[smem_spec, smem_spec, smem_spec, smem_spec,
                  pl.BlockSpec((_SC_WORKERS, _NS * _SC_L),
                               lambda: (0, 0)),
                  pl.BlockSpec((_K, 8, _LANES), lambda: (0, 0, 0)),
                  pl.BlockSpec((_K, 8, _LANES), lambda: (0, 0, 0))],
        out_specs=smem_spec,
        out_shape=jax.ShapeDtypeStruct((4,), jnp.float32),
    )(invs, scales, clips, base, scpart, top, bot)

  def _slow(xx):
    return pl.pallas_call(
        functools.partial(_sweep_exact_kernel, nsteps=nsteps),
        grid=(nsteps,),
        in_specs=[smem_spec, smem_spec, smem_spec,
                  pl.BlockSpec((_BLK_ROWS, _LANES), lambda i: (i, 0))],
        out_specs=smem_spec,
        out_shape=jax.ShapeDtypeStruct((4,), jnp.float32),
        scratch_shapes=[pltpu.SMEM((_NS,), jnp.float32)],
        compiler_params=pltpu.CompilerParams(
            dimension_semantics=("arbitrary",)),
    )(invs, scales, clips, xx)

  out = jax.lax.cond(risky, _slow, _fast, x)
  return (out[0].reshape(()), out[1].reshape(()),
          out[2].reshape(()), out[3].reshape(()))
